# baseline (speedup 1.0000x reference)
import functools

import jax
import jax.numpy as jnp
from jax import lax
from jax.experimental import pallas as pl
from jax.experimental.pallas import tpu as pltpu
from jax.experimental.pallas import tpu_sc as plsc

D = 128
CH = 80
DEPTH = 6
NC = 2
NS = 16
NW = NC * NS
L = 16
G = CH // L


def _sc_dot_gather(n_edges):
    assert n_edges % (DEPTH * CH * NW) == 0
    n_chunks = n_edges // (CH * NW)
    kah = DEPTH - 1

    mesh = plsc.VectorSubcoreMesh(core_axis_name="c", subcore_axis_name="s")

    @functools.partial(
        pl.kernel,
        mesh=mesh,
        compiler_params=pltpu.CompilerParams(needs_layout_passes=False),
        out_type=jax.ShapeDtypeStruct((n_edges,), jnp.float32),
        scratch_types=(
            [pltpu.VMEM((CH, D), jnp.float32) for _ in range(2 * DEPTH)]
            + [pltpu.VMEM((CH,), jnp.int32) for _ in range(2 * DEPTH)]
            + [pltpu.VMEM((CH,), jnp.float32) for _ in range(DEPTH)]
            + [pltpu.SemaphoreType.DMA for _ in range(4 * DEPTH)]
        ),
    )
    def k(xu, xb, iu, ib, out, *bufs):
        ru = list(bufs[0:DEPTH])
        rb = list(bufs[DEPTH:2 * DEPTH])
        ivu = list(bufs[2 * DEPTH:3 * DEPTH])
        ivb = list(bufs[3 * DEPTH:4 * DEPTH])
        ov = list(bufs[4 * DEPTH:5 * DEPTH])
        sems = bufs[5 * DEPTH:]
        su = list(sems[0:DEPTH])
        sb = list(sems[DEPTH:2 * DEPTH])
        qu = list(sems[2 * DEPTH:3 * DEPTH])
        qb = list(sems[3 * DEPTH:4 * DEPTH])

        wid = lax.axis_index("s") * NC + lax.axis_index("c")
        wbase = wid * n_chunks * CH
        iota = lax.iota(jnp.int32, L)
        rids = [g * L + iota for g in range(G)]

        def idx_copies(c, r):
            pltpu.make_async_copy(iu.at[pl.ds(wbase + c * CH, CH)], ivu[r], qu[r]).start()
            pltpu.make_async_copy(ib.at[pl.ds(wbase + c * CH, CH)], ivb[r], qb[r]).start()

        def wait_idx(c, r):
            pltpu.make_async_copy(iu.at[pl.ds(wbase + c * CH, CH)], ivu[r], qu[r]).wait()
            pltpu.make_async_copy(ib.at[pl.ds(wbase + c * CH, CH)], ivb[r], qb[r]).wait()

        def gathers(r):
            pltpu.make_async_copy(xu.at[ivu[r]], ru[r], su[r]).start()
            pltpu.make_async_copy(xb.at[ivb[r]], rb[r], sb[r]).start()

        def wait_gathers(r):
            pltpu.make_async_copy(xu.at[ivu[r]], ru[r], su[r]).wait()
            pltpu.make_async_copy(xb.at[ivb[r]], rb[r], sb[r]).wait()

        for c in range(kah + 1):
            idx_copies(c, c % DEPTH)
        for c in range(kah):
            wait_idx(c, c % DEPTH)
            gathers(c % DEPTH)

        def do_chunk(c, r):
            @pl.when(c + kah < n_chunks)
            def _():
                wait_idx(c + kah, (r + kah) % DEPTH)
                gathers((r + kah) % DEPTH)

            @pl.when(c + kah + 1 < n_chunks)
            def _():
                idx_copies(c + kah + 1, (r + kah + 1) % DEPTH)

            wait_gathers(r)

            def dbody(dd, accs):
                col = (iota + dd) & (D - 1)
                return tuple(
                    accs[g]
                    + plsc.load_gather(ru[r], [rids[g], col])
                    * plsc.load_gather(rb[r], [rids[g], col])
                    for g in range(G)
                )

            zero = jnp.zeros((L,), jnp.float32)
            accs = lax.fori_loop(0, D, dbody, tuple(zero for _ in range(G)))
            for g in range(G):
                ov[r][pl.ds(g * L, L)] = accs[g]
            pltpu.sync_copy(ov[r], out.at[pl.ds(wbase + c * CH, CH)])

        def ring_body(i, carry):
            for b in range(DEPTH):
                do_chunk(i * DEPTH + b, b)
            return carry

        lax.fori_loop(0, n_chunks // DEPTH, ring_body, 0)

    return k


def kernel(x_user, x_book, edge_label_index):
    eli = edge_label_index.astype(jnp.int32)
    n = eli.shape[1]
    step = DEPTH * CH * NW
    n_pad = ((n + step - 1) // step) * step
    iu = jnp.pad(eli[0], (0, n_pad - n))
    ib = jnp.pad(eli[1], (0, n_pad - n))
    out = _sc_dot_gather(n_pad)(x_user, x_book, iu, ib)
    return out[:n]


# speedup vs baseline: 1.9082x; 1.9082x over previous
import functools

import jax
import jax.numpy as jnp
from jax import lax
from jax.experimental import pallas as pl
from jax.experimental.pallas import tpu as pltpu
from jax.experimental.pallas import tpu_sc as plsc

D = 128
CH = 64
DEPTH = 7
NC = 2
NS = 16
NW = NC * NS
L = 16
G = CH // L


def _sc_dot_gather(n_edges):
    assert n_edges % (DEPTH * CH * NW) == 0
    n_chunks = n_edges // (CH * NW)
    kah = DEPTH - 1

    mesh = plsc.VectorSubcoreMesh(core_axis_name="c", subcore_axis_name="s")

    @functools.partial(
        pl.kernel,
        mesh=mesh,
        compiler_params=pltpu.CompilerParams(needs_layout_passes=False),
        out_type=jax.ShapeDtypeStruct((n_edges,), jnp.float32),
        scratch_types=(
            [pltpu.VMEM((CH, D), jnp.float32) for _ in range(2 * DEPTH)]
            + [pltpu.VMEM((CH,), jnp.int32) for _ in range(2 * DEPTH)]
            + [pltpu.VMEM((CH,), jnp.float32) for _ in range(DEPTH)]
            + [pltpu.SemaphoreType.DMA for _ in range(4 * DEPTH)]
        ),
    )
    def k(xu, xb, iu, ib, out, *bufs):
        ru = list(bufs[0:DEPTH])
        rb = list(bufs[DEPTH:2 * DEPTH])
        ivu = list(bufs[2 * DEPTH:3 * DEPTH])
        ivb = list(bufs[3 * DEPTH:4 * DEPTH])
        ov = list(bufs[4 * DEPTH:5 * DEPTH])
        sems = bufs[5 * DEPTH:]
        su = list(sems[0:DEPTH])
        sb = list(sems[DEPTH:2 * DEPTH])
        qu = list(sems[2 * DEPTH:3 * DEPTH])
        qb = list(sems[3 * DEPTH:4 * DEPTH])

        wid = lax.axis_index("s") * NC + lax.axis_index("c")
        wbase = wid * n_chunks * CH
        iota = lax.iota(jnp.int32, L)
        rids = [g * L + iota for g in range(G)]

        def idx_copies(c, r):
            pltpu.make_async_copy(iu.at[pl.ds(wbase + c * CH, CH)], ivu[r], qu[r]).start()
            pltpu.make_async_copy(ib.at[pl.ds(wbase + c * CH, CH)], ivb[r], qb[r]).start()

        def wait_idx(c, r):
            pltpu.make_async_copy(iu.at[pl.ds(wbase + c * CH, CH)], ivu[r], qu[r]).wait()
            pltpu.make_async_copy(ib.at[pl.ds(wbase + c * CH, CH)], ivb[r], qb[r]).wait()

        def gathers(r):
            pltpu.make_async_copy(xu.at[ivu[r]], ru[r], su[r]).start()
            pltpu.make_async_copy(xb.at[ivb[r]], rb[r], sb[r]).start()

        def wait_gathers(r):
            pltpu.make_async_copy(xu.at[ivu[r]], ru[r], su[r]).wait()
            pltpu.make_async_copy(xb.at[ivb[r]], rb[r], sb[r]).wait()

        for c in range(kah + 1):
            idx_copies(c, c % DEPTH)
        for c in range(kah):
            wait_idx(c, c % DEPTH)
            gathers(c % DEPTH)

        def do_chunk(c, r):
            @pl.when(c + kah < n_chunks)
            def _():
                wait_idx(c + kah, (r + kah) % DEPTH)
                gathers((r + kah) % DEPTH)

            @pl.when(c + kah + 1 < n_chunks)
            def _():
                idx_copies(c + kah + 1, (r + kah + 1) % DEPTH)

            wait_gathers(r)

            def dbody(dd, accs):
                col = (iota + dd) & (D - 1)
                return tuple(
                    accs[g]
                    + plsc.load_gather(ru[r], [rids[g], col])
                    * plsc.load_gather(rb[r], [rids[g], col])
                    for g in range(G)
                )

            zero = jnp.zeros((L,), jnp.float32)
            accs = lax.fori_loop(0, D, dbody, tuple(zero for _ in range(G)))
            for g in range(G):
                ov[r][pl.ds(g * L, L)] = accs[g]
            pltpu.sync_copy(ov[r], out.at[pl.ds(wbase + c * CH, CH)])

        def ring_body(i, carry):
            for b in range(DEPTH):
                do_chunk(i * DEPTH + b, b)
            return carry

        lax.fori_loop(0, n_chunks // DEPTH, ring_body, 0)

    return k


def kernel(x_user, x_book, edge_label_index):
    eli = edge_label_index.astype(jnp.int32)
    n = eli.shape[1]
    step = DEPTH * CH * NW
    n_pad = ((n + step - 1) // step) * step
    iu = jnp.pad(eli[0], (0, n_pad - n))
    ib = jnp.pad(eli[1], (0, n_pad - n))
    out = _sc_dot_gather(n_pad)(x_user, x_book, iu, ib)
    return out[:n]
